# baseline (speedup 1.0000x reference)
import functools

import jax
import jax.numpy as jnp
from jax import lax
from jax.experimental import pallas as pl
from jax.experimental.pallas import tpu as pltpu
from jax.experimental.pallas import tpu_sc as plsc

N_NODES = 10000
N_EDGES = 320000
NP = 10240
EP = 327680
EC = 128
NROWS = EP // EC
NC, NS = 2, 16
NW = NC * NS
CPW = NROWS // NW
RPT = NP // NS

D_IN = 128
D_HID = 64
D_O = 16


def _sc_mesh():
    return plsc.VectorSubcoreMesh(core_axis_name="c", subcore_axis_name="s")


@functools.partial(
    pl.kernel,
    out_type=jax.ShapeDtypeStruct((NC, NP, 8), jnp.float32),
    mesh=_sc_mesh(),
    scratch_types=[
        pltpu.VMEM_SHARED((NP, 8), jnp.float32),
        pltpu.VMEM((CPW, EC), jnp.int32),
        pltpu.VMEM((EC, 8), jnp.float32),
    ],
    compiler_params=pltpu.CompilerParams(use_tc_tiling_on_sc=False),
    name="deg_hist",
)
def _deg_kernel(dst2d, zeros_hbm, ones_hbm, out, acc, idx_v, ones_v):
    c = lax.axis_index("c")
    s = lax.axis_index("s")
    wid = s * NC + c
    pltpu.sync_copy(ones_hbm, ones_v)
    pltpu.sync_copy(dst2d.at[pl.ds(wid * CPW, CPW)], idx_v)
    pltpu.sync_copy(zeros_hbm, acc.at[pl.ds(s * RPT, RPT)])
    plsc.subcore_barrier()

    def body(j, carry):
        pltpu.sync_copy(ones_v, acc.at[idx_v.at[j]], add=True)
        return carry

    lax.fori_loop(0, CPW, body, 0)
    plsc.subcore_barrier()
    pltpu.sync_copy(acc.at[pl.ds(s * RPT, RPT)], out.at[c, pl.ds(s * RPT, RPT)])


def _make_spmm(d):
    @functools.partial(
        pl.kernel,
        out_type=jax.ShapeDtypeStruct((NC, NP, d), jnp.float32),
        mesh=_sc_mesh(),
        scratch_types=[
            pltpu.VMEM_SHARED((NP, d), jnp.float32),
            pltpu.VMEM((CPW, EC), jnp.int32),
            pltpu.VMEM((CPW, EC), jnp.int32),
            [pltpu.VMEM((EC, d), jnp.float32)] * 4,
            [pltpu.SemaphoreType.DMA] * 4,
            [pltpu.SemaphoreType.DMA] * 4,
        ],
        compiler_params=pltpu.CompilerParams(use_tc_tiling_on_sc=False),
        name=f"spmm{d}",
    )
    def spmm(table, src2d, dst2d, zeros_hbm, out, acc,
             src_v, dst_v, rows, gsem, ssem):
        c = lax.axis_index("c")
        s = lax.axis_index("s")
        wid = s * NC + c
        pltpu.sync_copy(src2d.at[pl.ds(wid * CPW, CPW)], src_v)
        pltpu.sync_copy(dst2d.at[pl.ds(wid * CPW, CPW)], dst_v)
        pltpu.sync_copy(zeros_hbm, acc.at[pl.ds(s * RPT, RPT)])
        plsc.subcore_barrier()

        for b in range(4):
            pltpu.async_copy(table.at[src_v.at[b]], rows[b], gsem[b])

        def body(k, carry):
            j = 4 * k
            for b in range(4):
                pltpu.make_async_copy(table.at[src_v.at[j + b]], rows[b], gsem[b]).wait()
                pltpu.async_copy(rows[b], acc.at[dst_v.at[j + b]], ssem[b], add=True)
            for b in range(4):
                jn = lax.rem(j + 4 + b, CPW)
                pltpu.make_async_copy(rows[b], acc.at[dst_v.at[j + b]], ssem[b]).wait()
                pltpu.async_copy(table.at[src_v.at[jn]], rows[b], gsem[b])
            return carry

        lax.fori_loop(0, CPW // 4, body, 0)
        for b in range(4):
            pltpu.make_async_copy(table.at[src_v.at[b]], rows[b], gsem[b]).wait()
        plsc.subcore_barrier()
        pltpu.sync_copy(acc.at[pl.ds(s * RPT, RPT)], out.at[c, pl.ds(s * RPT, RPT)])

    return spmm


_spmm64 = _make_spmm(D_HID)
_spmm16 = _make_spmm(D_O)


_BN = 512
_GRID = NP // _BN


def _dis(d0_ref, d1_ref):
    deg = d0_ref[:] + d1_ref[:] + 1.0
    return lax.rsqrt(deg)


def _tc1_body(x_ref, w_ref, d0_ref, d1_ref, o_ref):
    dis = _dis(d0_ref, d1_ref)
    h = jnp.dot(x_ref[:], w_ref[:], preferred_element_type=jnp.float32)
    o_ref[:] = h * dis[:, None]


def _tc2_body(a0_ref, a1_ref, h1_ref, d0_ref, d1_ref, w2_ref, b1_ref, o_ref):
    dis = _dis(d0_ref, d1_ref)
    z = dis[:, None] * (a0_ref[:] + a1_ref[:] + h1_ref[:]) + b1_ref[:]
    z = jnp.maximum(z, 0.0)
    h2 = jnp.dot(z, w2_ref[:], preferred_element_type=jnp.float32)
    o_ref[:] = h2 * dis[:, None]


def _tc3_body(a0_ref, a1_ref, h2_ref, d0_ref, d1_ref, b2_ref, o_ref):
    dis = _dis(d0_ref, d1_ref)
    o_ref[:] = dis[:, None] * (a0_ref[:] + a1_ref[:] + h2_ref[:]) + b2_ref[:]


def _row_spec(d):
    return pl.BlockSpec((_BN, d), lambda i: (i, 0))


def _vec_spec():
    return pl.BlockSpec((_BN,), lambda i: (i,))


def _full_spec(shape):
    return pl.BlockSpec(shape, lambda i: tuple(0 for _ in shape))


def _tc1(xp, W1, d0, d1):
    return pl.pallas_call(
        _tc1_body,
        grid=(_GRID,),
        in_specs=[_row_spec(D_IN), _full_spec((D_IN, D_HID)), _vec_spec(), _vec_spec()],
        out_specs=_row_spec(D_HID),
        out_shape=jax.ShapeDtypeStruct((NP, D_HID), jnp.float32),
    )(xp, W1, d0, d1)


def _tc2(a0, a1, h1p, d0, d1, W2p, b1):
    return pl.pallas_call(
        _tc2_body,
        grid=(_GRID,),
        in_specs=[
            _row_spec(D_HID), _row_spec(D_HID), _row_spec(D_HID),
            _vec_spec(), _vec_spec(),
            _full_spec((D_HID, D_O)), _full_spec((1, D_HID)),
        ],
        out_specs=_row_spec(D_O),
        out_shape=jax.ShapeDtypeStruct((NP, D_O), jnp.float32),
    )(a0, a1, h1p, d0, d1, W2p, b1)


def _tc3(a0, a1, h2p, d0, d1, b2p):
    return pl.pallas_call(
        _tc3_body,
        grid=(_GRID,),
        in_specs=[
            _row_spec(D_O), _row_spec(D_O), _row_spec(D_O),
            _vec_spec(), _vec_spec(),
            _full_spec((1, D_O)),
        ],
        out_specs=_row_spec(D_O),
        out_shape=jax.ShapeDtypeStruct((NP, D_O), jnp.float32),
    )(a0, a1, h2p, d0, d1, b2p)


def kernel(x, edge_index, W1, b1, W2, b2):
    src = jnp.asarray(edge_index[0], jnp.int32)
    dst = jnp.asarray(edge_index[1], jnp.int32)
    src2d = jnp.pad(src, (0, EP - N_EDGES)).reshape(NROWS, EC)
    dst2d = jnp.pad(dst, (0, EP - N_EDGES), constant_values=N_NODES).reshape(NROWS, EC)
    xp = jnp.pad(x, ((0, NP - N_NODES), (0, 0)))
    W2p = jnp.pad(W2, ((0, 0), (0, D_O - W2.shape[1])))
    b2p = jnp.pad(b2, (0, D_O - b2.shape[0])).reshape(1, D_O)
    b1r = b1.reshape(1, D_HID)

    z8 = jnp.zeros((RPT, 8), jnp.float32)
    z64 = jnp.zeros((RPT, D_HID), jnp.float32)
    z16 = jnp.zeros((RPT, D_O), jnp.float32)
    ones8 = jnp.ones((EC, 8), jnp.float32)

    deg_pair = _deg_kernel(dst2d, z8, ones8)
    d0 = deg_pair[0, :, 0]
    d1 = deg_pair[1, :, 0]

    h1p = _tc1(xp, W1, d0, d1)
    acc1 = _spmm64(h1p, src2d, dst2d, z64)
    h2p = _tc2(acc1[0], acc1[1], h1p, d0, d1, W2p, b1r)
    acc2 = _spmm16(h2p, src2d, dst2d, z16)
    outp = _tc3(acc2[0], acc2[1], h2p, d0, d1, b2p)
    return outp[:N_NODES, :2]


# speedup vs baseline: 1.0219x; 1.0219x over previous
import functools

import jax
import jax.numpy as jnp
from jax import lax
from jax.experimental import pallas as pl
from jax.experimental.pallas import tpu as pltpu
from jax.experimental.pallas import tpu_sc as plsc

N_NODES = 10000
N_EDGES = 320000
NP = 10240
EP = 327680
EC = 128
NROWS = EP // EC
NC, NS = 2, 16
NW = NC * NS
CPW = NROWS // NW
RPT = NP // NS

D_IN = 128
D_HID = 64
D_O = 16


def _sc_mesh():
    return plsc.VectorSubcoreMesh(core_axis_name="c", subcore_axis_name="s")


@functools.partial(
    pl.kernel,
    out_type=jax.ShapeDtypeStruct((NC, NP, 8), jnp.float32),
    mesh=_sc_mesh(),
    scratch_types=[
        pltpu.VMEM_SHARED((NP, 8), jnp.float32),
        pltpu.VMEM((CPW, EC), jnp.int32),
        pltpu.VMEM((EC, 8), jnp.float32),
    ],
    compiler_params=pltpu.CompilerParams(use_tc_tiling_on_sc=False),
    name="deg_hist",
)
def _deg_kernel(dst2d, zeros_hbm, ones_hbm, out, acc, idx_v, ones_v):
    c = lax.axis_index("c")
    s = lax.axis_index("s")
    wid = s * NC + c
    pltpu.sync_copy(ones_hbm, ones_v)
    pltpu.sync_copy(dst2d.at[pl.ds(wid * CPW, CPW)], idx_v)
    pltpu.sync_copy(zeros_hbm, acc.at[pl.ds(s * RPT, RPT)])
    plsc.subcore_barrier()

    def body(j, carry):
        pltpu.sync_copy(ones_v, acc.at[idx_v.at[j]], add=True)
        return carry

    lax.fori_loop(0, CPW, body, 0)
    plsc.subcore_barrier()
    pltpu.sync_copy(acc.at[pl.ds(s * RPT, RPT)], out.at[c, pl.ds(s * RPT, RPT)])


def _make_spmm(d):
    @functools.partial(
        pl.kernel,
        out_type=jax.ShapeDtypeStruct((NC, NP, d), jnp.float32),
        mesh=_sc_mesh(),
        scratch_types=[
            pltpu.VMEM_SHARED((NP, d), jnp.float32),
            pltpu.VMEM((CPW, EC), jnp.int32),
            pltpu.VMEM((CPW, EC), jnp.int32),
            [pltpu.VMEM((EC, d), jnp.float32)] * 4,
            [pltpu.SemaphoreType.DMA] * 4,
            [pltpu.SemaphoreType.DMA] * 4,
        ],
        compiler_params=pltpu.CompilerParams(use_tc_tiling_on_sc=False),
        name=f"spmm{d}",
    )
    def spmm(table, src2d, dst2d, zeros_hbm, out, acc,
             src_v, dst_v, rows, gsem, ssem):
        c = lax.axis_index("c")
        s = lax.axis_index("s")
        wid = s * NC + c
        pltpu.sync_copy(src2d.at[pl.ds(wid * CPW, CPW)], src_v)
        pltpu.sync_copy(dst2d.at[pl.ds(wid * CPW, CPW)], dst_v)
        pltpu.sync_copy(zeros_hbm, acc.at[pl.ds(s * RPT, RPT)])
        plsc.subcore_barrier()

        for b in range(4):
            pltpu.async_copy(table.at[src_v.at[b]], rows[b], gsem[b])

        def body(k, carry):
            j = 4 * k
            for b in range(4):
                pltpu.make_async_copy(table.at[src_v.at[j + b]], rows[b], gsem[b]).wait()
                pltpu.sync_copy(rows[b], acc.at[dst_v.at[j + b]], add=True)
                jn = lax.rem(j + 4 + b, CPW)
                pltpu.async_copy(table.at[src_v.at[jn]], rows[b], gsem[b])
            return carry

        lax.fori_loop(0, CPW // 4, body, 0)
        for b in range(4):
            pltpu.make_async_copy(table.at[src_v.at[b]], rows[b], gsem[b]).wait()
        plsc.subcore_barrier()
        pltpu.sync_copy(acc.at[pl.ds(s * RPT, RPT)], out.at[c, pl.ds(s * RPT, RPT)])

    return spmm


_spmm64 = _make_spmm(D_HID)
_spmm16 = _make_spmm(D_O)


_BN = 512
_GRID = NP // _BN


def _dis(d0_ref, d1_ref):
    deg = d0_ref[:] + d1_ref[:] + 1.0
    return lax.rsqrt(deg)


def _tc1_body(x_ref, w_ref, d0_ref, d1_ref, o_ref):
    dis = _dis(d0_ref, d1_ref)
    h = jnp.dot(x_ref[:], w_ref[:], preferred_element_type=jnp.float32)
    o_ref[:] = h * dis[:, None]


def _tc2_body(a0_ref, a1_ref, h1_ref, d0_ref, d1_ref, w2_ref, b1_ref, o_ref):
    dis = _dis(d0_ref, d1_ref)
    z = dis[:, None] * (a0_ref[:] + a1_ref[:] + h1_ref[:]) + b1_ref[:]
    z = jnp.maximum(z, 0.0)
    h2 = jnp.dot(z, w2_ref[:], preferred_element_type=jnp.float32)
    o_ref[:] = h2 * dis[:, None]


def _tc3_body(a0_ref, a1_ref, h2_ref, d0_ref, d1_ref, b2_ref, o_ref):
    dis = _dis(d0_ref, d1_ref)
    o_ref[:] = dis[:, None] * (a0_ref[:] + a1_ref[:] + h2_ref[:]) + b2_ref[:]


def _row_spec(d):
    return pl.BlockSpec((_BN, d), lambda i: (i, 0))


def _vec_spec():
    return pl.BlockSpec((_BN,), lambda i: (i,))


def _full_spec(shape):
    return pl.BlockSpec(shape, lambda i: tuple(0 for _ in shape))


def _tc1(xp, W1, d0, d1):
    return pl.pallas_call(
        _tc1_body,
        grid=(_GRID,),
        in_specs=[_row_spec(D_IN), _full_spec((D_IN, D_HID)), _vec_spec(), _vec_spec()],
        out_specs=_row_spec(D_HID),
        out_shape=jax.ShapeDtypeStruct((NP, D_HID), jnp.float32),
    )(xp, W1, d0, d1)


def _tc2(a0, a1, h1p, d0, d1, W2p, b1):
    return pl.pallas_call(
        _tc2_body,
        grid=(_GRID,),
        in_specs=[
            _row_spec(D_HID), _row_spec(D_HID), _row_spec(D_HID),
            _vec_spec(), _vec_spec(),
            _full_spec((D_HID, D_O)), _full_spec((1, D_HID)),
        ],
        out_specs=_row_spec(D_O),
        out_shape=jax.ShapeDtypeStruct((NP, D_O), jnp.float32),
    )(a0, a1, h1p, d0, d1, W2p, b1)


def _tc3(a0, a1, h2p, d0, d1, b2p):
    return pl.pallas_call(
        _tc3_body,
        grid=(_GRID,),
        in_specs=[
            _row_spec(D_O), _row_spec(D_O), _row_spec(D_O),
            _vec_spec(), _vec_spec(),
            _full_spec((1, D_O)),
        ],
        out_specs=_row_spec(D_O),
        out_shape=jax.ShapeDtypeStruct((NP, D_O), jnp.float32),
    )(a0, a1, h2p, d0, d1, b2p)


def kernel(x, edge_index, W1, b1, W2, b2):
    src = jnp.asarray(edge_index[0], jnp.int32)
    dst = jnp.asarray(edge_index[1], jnp.int32)
    src2d = jnp.pad(src, (0, EP - N_EDGES)).reshape(NROWS, EC)
    dst2d = jnp.pad(dst, (0, EP - N_EDGES), constant_values=N_NODES).reshape(NROWS, EC)
    xp = jnp.pad(x, ((0, NP - N_NODES), (0, 0)))
    W2p = jnp.pad(W2, ((0, 0), (0, D_O - W2.shape[1])))
    b2p = jnp.pad(b2, (0, D_O - b2.shape[0])).reshape(1, D_O)
    b1r = b1.reshape(1, D_HID)

    z8 = jnp.zeros((RPT, 8), jnp.float32)
    z64 = jnp.zeros((RPT, D_HID), jnp.float32)
    z16 = jnp.zeros((RPT, D_O), jnp.float32)
    ones8 = jnp.ones((EC, 8), jnp.float32)

    deg_pair = _deg_kernel(dst2d, z8, ones8)
    d0 = deg_pair[0, :, 0]
    d1 = deg_pair[1, :, 0]

    h1p = _tc1(xp, W1, d0, d1)
    acc1 = _spmm64(h1p, src2d, dst2d, z64)
    h2p = _tc2(acc1[0], acc1[1], h1p, d0, d1, W2p, b1r)
    acc2 = _spmm16(h2p, src2d, dst2d, z16)
    outp = _tc3(acc2[0], acc2[1], h2p, d0, d1, b2p)
    return outp[:N_NODES, :2]


# speedup vs baseline: 1.8190x; 1.7801x over previous
import functools

import jax
import jax.numpy as jnp
from jax import lax
from jax.experimental import pallas as pl
from jax.experimental.pallas import tpu as pltpu
from jax.experimental.pallas import tpu_sc as plsc

N_NODES = 10000
N_EDGES = 320000
NP = 10240
EP = 327680
EC = 128
NROWS = EP // EC
NC, NS = 2, 16
NW = NC * NS
CPW = NROWS // NW
RPT = NP // NS

D_IN = 128
D_HID = 64
D_O = 16


def _sc_mesh():
    return plsc.VectorSubcoreMesh(core_axis_name="c", subcore_axis_name="s")


@functools.partial(
    pl.kernel,
    out_type=jax.ShapeDtypeStruct((NC, NP, 8), jnp.float32),
    mesh=_sc_mesh(),
    scratch_types=[
        pltpu.VMEM_SHARED((NP, 8), jnp.float32),
        pltpu.VMEM((CPW, EC), jnp.int32),
        pltpu.VMEM((EC, 8), jnp.float32),
    ],
    compiler_params=pltpu.CompilerParams(use_tc_tiling_on_sc=False),
    name="deg_hist",
)
def _deg_kernel(dst2d, zeros_hbm, ones_hbm, out, acc, idx_v, ones_v):
    c = lax.axis_index("c")
    s = lax.axis_index("s")
    wid = s * NC + c
    pltpu.sync_copy(ones_hbm, ones_v)
    pltpu.sync_copy(dst2d.at[pl.ds(wid * CPW, CPW)], idx_v)
    pltpu.sync_copy(zeros_hbm, acc.at[pl.ds(s * RPT, RPT)])
    plsc.subcore_barrier()

    def body(j, carry):
        pltpu.sync_copy(ones_v, acc.at[idx_v.at[j]], add=True)
        return carry

    lax.fori_loop(0, CPW, body, 0)
    plsc.subcore_barrier()
    pltpu.sync_copy(acc.at[pl.ds(s * RPT, RPT)], out.at[c, pl.ds(s * RPT, RPT)])


def _make_spmm(d):
    nbuf = 2 if d == D_HID else 4
    @functools.partial(
        pl.kernel,
        out_type=jax.ShapeDtypeStruct((NC, NP, d), jnp.float32),
        mesh=_sc_mesh(),
        scratch_types=[
            pltpu.VMEM_SHARED((NP, d), jnp.float32),
            pltpu.VMEM_SHARED((NP, d), jnp.float32),
            pltpu.VMEM((CPW, EC), jnp.int32),
            pltpu.VMEM((CPW, EC), jnp.int32),
            [pltpu.VMEM((EC, d), jnp.float32)] * nbuf,
            [pltpu.SemaphoreType.DMA] * nbuf,
        ],
        compiler_params=pltpu.CompilerParams(use_tc_tiling_on_sc=False),
        name=f"spmm{d}",
    )
    def spmm(table, src2d, dst2d, zeros_hbm, out, acc, table_s,
             src_v, dst_v, rows, gsem):
        c = lax.axis_index("c")
        s = lax.axis_index("s")
        wid = s * NC + c
        pltpu.sync_copy(src2d.at[pl.ds(wid * CPW, CPW)], src_v)
        pltpu.sync_copy(dst2d.at[pl.ds(wid * CPW, CPW)], dst_v)
        pltpu.sync_copy(zeros_hbm, acc.at[pl.ds(s * RPT, RPT)])
        pltpu.sync_copy(table.at[pl.ds(s * RPT, RPT)], table_s.at[pl.ds(s * RPT, RPT)])
        plsc.subcore_barrier()

        for b in range(nbuf):
            pltpu.async_copy(table_s.at[src_v.at[b]], rows[b], gsem[b])

        def body(k, carry):
            j = nbuf * k
            for b in range(nbuf):
                pltpu.make_async_copy(table_s.at[src_v.at[j + b]], rows[b], gsem[b]).wait()
                pltpu.sync_copy(rows[b], acc.at[dst_v.at[j + b]], add=True)
                jn = lax.rem(j + 4 + b, CPW)
                pltpu.async_copy(table_s.at[src_v.at[jn]], rows[b], gsem[b])
            return carry

        lax.fori_loop(0, CPW // nbuf, body, 0)
        for b in range(nbuf):
            pltpu.make_async_copy(table_s.at[src_v.at[b]], rows[b], gsem[b]).wait()
        plsc.subcore_barrier()
        pltpu.sync_copy(acc.at[pl.ds(s * RPT, RPT)], out.at[c, pl.ds(s * RPT, RPT)])

    return spmm


_spmm64 = _make_spmm(D_HID)
_spmm16 = _make_spmm(D_O)


_BN = 512
_GRID = NP // _BN


def _dis(d0_ref, d1_ref):
    deg = d0_ref[:] + d1_ref[:] + 1.0
    return lax.rsqrt(deg)


def _tc1_body(x_ref, w_ref, d0_ref, d1_ref, o_ref):
    dis = _dis(d0_ref, d1_ref)
    h = jnp.dot(x_ref[:], w_ref[:], preferred_element_type=jnp.float32)
    o_ref[:] = h * dis[:, None]


def _tc2_body(a0_ref, a1_ref, h1_ref, d0_ref, d1_ref, w2_ref, b1_ref, o_ref):
    dis = _dis(d0_ref, d1_ref)
    z = dis[:, None] * (a0_ref[:] + a1_ref[:] + h1_ref[:]) + b1_ref[:]
    z = jnp.maximum(z, 0.0)
    h2 = jnp.dot(z, w2_ref[:], preferred_element_type=jnp.float32)
    o_ref[:] = h2 * dis[:, None]


def _tc3_body(a0_ref, a1_ref, h2_ref, d0_ref, d1_ref, b2_ref, o_ref):
    dis = _dis(d0_ref, d1_ref)
    o_ref[:] = dis[:, None] * (a0_ref[:] + a1_ref[:] + h2_ref[:]) + b2_ref[:]


def _row_spec(d):
    return pl.BlockSpec((_BN, d), lambda i: (i, 0))


def _vec_spec():
    return pl.BlockSpec((_BN,), lambda i: (i,))


def _full_spec(shape):
    return pl.BlockSpec(shape, lambda i: tuple(0 for _ in shape))


def _tc1(xp, W1, d0, d1):
    return pl.pallas_call(
        _tc1_body,
        grid=(_GRID,),
        in_specs=[_row_spec(D_IN), _full_spec((D_IN, D_HID)), _vec_spec(), _vec_spec()],
        out_specs=_row_spec(D_HID),
        out_shape=jax.ShapeDtypeStruct((NP, D_HID), jnp.float32),
    )(xp, W1, d0, d1)


def _tc2(a0, a1, h1p, d0, d1, W2p, b1):
    return pl.pallas_call(
        _tc2_body,
        grid=(_GRID,),
        in_specs=[
            _row_spec(D_HID), _row_spec(D_HID), _row_spec(D_HID),
            _vec_spec(), _vec_spec(),
            _full_spec((D_HID, D_O)), _full_spec((1, D_HID)),
        ],
        out_specs=_row_spec(D_O),
        out_shape=jax.ShapeDtypeStruct((NP, D_O), jnp.float32),
    )(a0, a1, h1p, d0, d1, W2p, b1)


def _tc3(a0, a1, h2p, d0, d1, b2p):
    return pl.pallas_call(
        _tc3_body,
        grid=(_GRID,),
        in_specs=[
            _row_spec(D_O), _row_spec(D_O), _row_spec(D_O),
            _vec_spec(), _vec_spec(),
            _full_spec((1, D_O)),
        ],
        out_specs=_row_spec(D_O),
        out_shape=jax.ShapeDtypeStruct((NP, D_O), jnp.float32),
    )(a0, a1, h2p, d0, d1, b2p)


def kernel(x, edge_index, W1, b1, W2, b2):
    src = jnp.asarray(edge_index[0], jnp.int32)
    dst = jnp.asarray(edge_index[1], jnp.int32)
    src2d = jnp.pad(src, (0, EP - N_EDGES)).reshape(NROWS, EC)
    dst2d = jnp.pad(dst, (0, EP - N_EDGES), constant_values=N_NODES).reshape(NROWS, EC)
    xp = jnp.pad(x, ((0, NP - N_NODES), (0, 0)))
    W2p = jnp.pad(W2, ((0, 0), (0, D_O - W2.shape[1])))
    b2p = jnp.pad(b2, (0, D_O - b2.shape[0])).reshape(1, D_O)
    b1r = b1.reshape(1, D_HID)

    z8 = jnp.zeros((RPT, 8), jnp.float32)
    z64 = jnp.zeros((RPT, D_HID), jnp.float32)
    z16 = jnp.zeros((RPT, D_O), jnp.float32)
    ones8 = jnp.ones((EC, 8), jnp.float32)

    deg_pair = _deg_kernel(dst2d, z8, ones8)
    d0 = deg_pair[0, :, 0]
    d1 = deg_pair[1, :, 0]

    h1p = _tc1(xp, W1, d0, d1)
    acc1 = _spmm64(h1p, src2d, dst2d, z64)
    h2p = _tc2(acc1[0], acc1[1], h1p, d0, d1, W2p, b1r)
    acc2 = _spmm16(h2p, src2d, dst2d, z16)
    outp = _tc3(acc2[0], acc2[1], h2p, d0, d1, b2p)
    return outp[:N_NODES, :2]
